# baseline (speedup 1.0000x reference)
import functools

import jax
import jax.numpy as jnp
from jax import lax
from jax.experimental import pallas as pl
from jax.experimental.pallas import tpu as pltpu
from jax.experimental.pallas import tpu_sc as plsc

N = 10000
E = 320000
D = 128

NC = 2
NS = 16
NW = NC * NS
EPW = E // NW
CHUNK = 80
NCHUNK = EPW // CHUNK
NPAD = 10240
RPS = NPAD // NS

@functools.cache
def _mesh():
  return plsc.VectorSubcoreMesh(
      core_axis_name="c", subcore_axis_name="s", num_cores=NC, num_subcores=NS)


def _wid():
  return lax.axis_index("c") * NS + lax.axis_index("s")


def _vgather(vec, idx):
  return lax.gather(
      vec, idx[:, None],
      dimension_numbers=lax.GatherDimensionNumbers(
          offset_dims=(), collapsed_slice_dims=(0,), start_index_map=(0,)),
      slice_sizes=(1,),
      mode=lax.GatherScatterMode.PROMISE_IN_BOUNDS)


def _ew_deg_body(ef0_hbm, ef1_hbm, ef2_hbm, ef3_hbm, col3_hbm, ew_hbm, deg_hbm,
                 f0_v, f1_v, f2_v, f3_v, ew_v, col_v, z_v, deg_sp):
  cid = lax.axis_index("c")
  sid = lax.axis_index("s")
  wid = _wid()
  base = pl.multiple_of(wid * EPW, EPW)

  def zb(i, _):
    z_v[pl.ds(i * 16, 16)] = jnp.zeros((16,), jnp.float32)
    return 0
  lax.fori_loop(0, RPS // 16, zb, 0)
  pltpu.sync_copy(z_v, deg_sp.at[pl.ds(sid * RPS, RPS)])

  pltpu.sync_copy(col3_hbm.at[wid], col_v)
  pltpu.sync_copy(ef0_hbm.at[pl.ds(base, EPW)], f0_v)
  pltpu.sync_copy(ef1_hbm.at[pl.ds(base, EPW)], f1_v)
  pltpu.sync_copy(ef2_hbm.at[pl.ds(base, EPW)], f2_v)
  pltpu.sync_copy(ef3_hbm.at[pl.ds(base, EPW)], f3_v)

  def ewb(t, _):
    e0 = t * 16
    s = (f0_v[pl.ds(e0, 16)] + f1_v[pl.ds(e0, 16)]
         + f2_v[pl.ds(e0, 16)] + f3_v[pl.ds(e0, 16)])
    ew_v[pl.ds(e0, 16)] = s * 0.25
    return 0
  lax.fori_loop(0, EPW // 16, ewb, 0)
  pltpu.sync_copy(ew_v, ew_hbm.at[pl.ds(base, EPW)])

  plsc.subcore_barrier()

  def db(j, _):
    pltpu.sync_copy(ew_v.at[pl.ds(pl.multiple_of(j * CHUNK, CHUNK), CHUNK)],
                    deg_sp.at[col_v.at[j]], add=True)
    return 0
  lax.fori_loop(0, NCHUNK, db, 0)

  plsc.subcore_barrier()
  pltpu.sync_copy(deg_sp.at[pl.ds(sid * RPS, RPS)],
                  deg_hbm.at[cid, pl.ds(sid * RPS, RPS)])


@functools.cache
def _ew_deg():
  return pl.kernel(
    _ew_deg_body,
    out_type=(jax.ShapeDtypeStruct((E,), jnp.float32),
              jax.ShapeDtypeStruct((NC, NPAD), jnp.float32)),
    mesh=_mesh(),
    scratch_types=[
        pltpu.VMEM((EPW,), jnp.float32),
        pltpu.VMEM((EPW,), jnp.float32),
        pltpu.VMEM((EPW,), jnp.float32),
        pltpu.VMEM((EPW,), jnp.float32),
        pltpu.VMEM((EPW,), jnp.float32),
        pltpu.VMEM((NCHUNK, CHUNK), jnp.int32),
        pltpu.VMEM((RPS,), jnp.float32),
        pltpu.VMEM_SHARED((NPAD,), jnp.float32),
    ],
    compiler_params=pltpu.CompilerParams(needs_layout_passes=False),
  )


def _agg_body(g_hbm, row_hbm, col_hbm, ew_hbm, acc_hbm, *refs):
  row_s = refs[0:4]
  col_s = refs[4:8]
  ew_s = refs[8:12]
  rows = refs[12:16]
  acc_sp = refs[16]
  isem = refs[17:21]
  gsem = refs[21:25]
  ssem = refs[25:29]
  cid = lax.axis_index("c")
  sid = lax.axis_index("s")
  wid = _wid()
  base = pl.multiple_of(wid * EPW, EPW)

  def zb(i, _):
    r = i // 8
    j = i % 8
    rows[0][r, pl.ds(j * 16, 16)] = jnp.zeros((16,), jnp.float32)
    return 0
  lax.fori_loop(0, CHUNK * 8, zb, 0)
  for k in range(RPS // CHUNK):
    pltpu.sync_copy(rows[0], acc_sp.at[pl.ds(sid * RPS + k * CHUNK, CHUNK)])
  plsc.subcore_barrier()

  def scale(t):
    @plsc.parallel_loop(0, CHUNK // 16, unroll=1)
    def _(g):
      ew16 = ew_s[t][pl.ds(g * 16, 16)]
      for e in range(16):
        bc = _vgather(ew16, jnp.full((16,), e, jnp.int32))
        r = g * 16 + e
        for jj in range(D // 16):
          rows[t][r, pl.ds(jj * 16, 16)] = rows[t][r, pl.ds(jj * 16, 16)] * bc

  def idx_issue(j, t):
    off = pl.multiple_of(base + j * CHUNK, CHUNK)
    pltpu.async_copy(row_hbm.at[pl.ds(off, CHUNK)], row_s[t], isem[t])
    pltpu.async_copy(col_hbm.at[pl.ds(off, CHUNK)], col_s[t], isem[t])
    pltpu.async_copy(ew_hbm.at[pl.ds(off, CHUNK)], ew_s[t], isem[t])

  def idx_wait(t):
    pltpu.make_async_copy(row_hbm.at[pl.ds(base, CHUNK)], row_s[t],
                          isem[t]).wait()
    pltpu.make_async_copy(col_hbm.at[pl.ds(base, CHUNK)], col_s[t],
                          isem[t]).wait()
    pltpu.make_async_copy(ew_hbm.at[pl.ds(base, CHUNK)], ew_s[t],
                          isem[t]).wait()

  def gissue(j, t):
    pltpu.async_copy(g_hbm.at[row_s[t]], rows[t], gsem[t])

  def gwait(t):
    pltpu.make_async_copy(g_hbm.at[row_s[t]], rows[t], gsem[t]).wait()

  def sissue(t):
    pltpu.async_copy(rows[t], acc_sp.at[col_s[t]], ssem[t], add=True)

  def swait(t):
    pltpu.make_async_copy(rows[t], acc_sp.at[col_s[t]], ssem[t]).wait()

  idx_issue(0, 0)
  idx_issue(1, 1)
  idx_wait(0)
  gissue(0, 0)

  idx_wait(1)
  gissue(1, 1)
  gwait(0)
  scale(0)
  sissue(0)
  idx_issue(2, 2)
  idx_wait(2)
  gissue(2, 2)
  gwait(1)
  scale(1)
  sissue(1)
  idx_issue(3, 3)

  def quad(i, _):
    for u in range(4):
      j = 2 + 4 * i + u
      t = (2 + u) % 4
      t1 = (3 + u) % 4
      t2 = u
      idx_wait(t1)
      gissue(j + 1, t1)
      gwait(t)
      scale(t)
      sissue(t)
      swait(t2)
      idx_issue(j + 2, t2)
    return 0
  lax.fori_loop(0, (NCHUNK - 5) // 4, quad, 0)

  idx_wait(3)
  gissue(123, 3)
  gwait(2)
  scale(2)
  sissue(2)
  swait(0)
  idx_issue(124, 0)

  idx_wait(0)
  gissue(124, 0)
  gwait(3)
  scale(3)
  sissue(3)
  swait(1)

  gwait(0)
  scale(0)
  sissue(0)
  swait(2)
  swait(3)
  swait(0)

  plsc.subcore_barrier()
  for k in range(RPS // CHUNK):
    pltpu.sync_copy(acc_sp.at[pl.ds(sid * RPS + k * CHUNK, CHUNK)],
                    acc_hbm.at[cid, pl.ds(sid * RPS + k * CHUNK, CHUNK)])


@functools.cache
def _agg():
  return pl.kernel(
    _agg_body,
    out_type=jax.ShapeDtypeStruct((NC, NPAD, D), jnp.float32),
    mesh=_mesh(),
    scratch_types=(
        [pltpu.VMEM((CHUNK,), jnp.int32)] * 4
        + [pltpu.VMEM((CHUNK,), jnp.int32)] * 4
        + [pltpu.VMEM((CHUNK,), jnp.float32)] * 4
        + [pltpu.VMEM((CHUNK, D), jnp.float32)] * 4
        + [pltpu.VMEM_SHARED((NPAD, D), jnp.float32)]
        + [pltpu.SemaphoreType.DMA] * 12
    ),
    compiler_params=pltpu.CompilerParams(needs_layout_passes=False),
  )


BN = 1000


def _tc1_body(x_ref, w_ref, degp_ref, dis_ref, g_ref):
  d = degp_ref[0] + degp_ref[1] + 1.0
  dis = jnp.where(d > 0, lax.rsqrt(d), 0.0)
  dis_ref[...] = dis
  h = jnp.dot(x_ref[...], w_ref[...], preferred_element_type=jnp.float32)
  g_ref[...] = h * dis


def _tc1(x, w, degp):
  return pl.pallas_call(
      _tc1_body,
      grid=(N // BN,),
      in_specs=[
          pl.BlockSpec((BN, D), lambda i: (i, 0)),
          pl.BlockSpec((D, D), lambda i: (0, 0)),
          pl.BlockSpec((NC, BN, 1), lambda i: (0, i, 0)),
      ],
      out_specs=[
          pl.BlockSpec((BN, 1), lambda i: (i, 0)),
          pl.BlockSpec((BN, D), lambda i: (i, 0)),
      ],
      out_shape=[
          jax.ShapeDtypeStruct((N, 1), jnp.float32),
          jax.ShapeDtypeStruct((N, D), jnp.float32),
      ],
  )(x, w, degp)


def _tc_mid_body(accp_ref, g_ref, dis_ref, b_ref, w_ref, g2_ref):
  acc = accp_ref[0] + accp_ref[1]
  out = dis_ref[...] * (acc + g_ref[...]) + b_ref[...]
  h2 = jnp.maximum(out, 0.0)
  g2_ref[...] = jnp.dot(h2, w_ref[...],
                        preferred_element_type=jnp.float32) * dis_ref[...]


def _tc_mid(accp, g, dis, b, w):
  return pl.pallas_call(
      _tc_mid_body,
      grid=(N // BN,),
      in_specs=[
          pl.BlockSpec((NC, BN, D), lambda i: (0, i, 0)),
          pl.BlockSpec((BN, D), lambda i: (i, 0)),
          pl.BlockSpec((BN, 1), lambda i: (i, 0)),
          pl.BlockSpec((1, D), lambda i: (0, 0)),
          pl.BlockSpec((D, D), lambda i: (0, 0)),
      ],
      out_specs=pl.BlockSpec((BN, D), lambda i: (i, 0)),
      out_shape=jax.ShapeDtypeStruct((N, D), jnp.float32),
  )(accp, g, dis, b, w)


def _tc_fin_body(accp_ref, g_ref, dis_ref, b_ref, out_ref):
  acc = accp_ref[0] + accp_ref[1]
  out_ref[...] = dis_ref[...] * (acc + g_ref[...]) + b_ref[...]


def _tc_fin(accp, g, dis, b):
  return pl.pallas_call(
      _tc_fin_body,
      grid=(N // BN,),
      in_specs=[
          pl.BlockSpec((NC, BN, D), lambda i: (0, i, 0)),
          pl.BlockSpec((BN, D), lambda i: (i, 0)),
          pl.BlockSpec((BN, 1), lambda i: (i, 0)),
          pl.BlockSpec((1, D), lambda i: (0, 0)),
      ],
      out_specs=pl.BlockSpec((BN, D), lambda i: (i, 0)),
      out_shape=jax.ShapeDtypeStruct((N, D), jnp.float32),
  )(accp, g, dis, b)


def kernel(x, edge_index, edge_feature, W1, b1, W2, b2):
  row = edge_index[0]
  col = edge_index[1]
  col3 = col.reshape(NW, NCHUNK, CHUNK)
  eft = edge_feature.T

  ew, deg = _ew_deg()(eft[0], eft[1], eft[2], eft[3], col3)
  degp = deg[:, :N].reshape(NC, N, 1)
  dis, g1 = _tc1(x, W1, degp)

  acc1 = _agg()(g1, row, col, ew)
  g2 = _tc_mid(acc1, g1, dis, b1.reshape(1, D), W2)

  acc2 = _agg()(g2, row, col, ew)
  out = _tc_fin(acc2, g2, dis, b2.reshape(1, D))
  return out


# speedup vs baseline: 1.1687x; 1.1687x over previous
import functools

import jax
import jax.numpy as jnp
from jax import lax
from jax.experimental import pallas as pl
from jax.experimental.pallas import tpu as pltpu
from jax.experimental.pallas import tpu_sc as plsc

N = 10000
E = 320000
D = 128

NC = 2
NS = 16
NW = NC * NS
EPW = E // NW
CHUNK = 80
NCHUNK = EPW // CHUNK
NPAD = 10240
RPS = NPAD // NS

@functools.cache
def _mesh():
  return plsc.VectorSubcoreMesh(
      core_axis_name="c", subcore_axis_name="s", num_cores=NC, num_subcores=NS)


def _wid():
  return lax.axis_index("c") * NS + lax.axis_index("s")


def _ew_deg_body(ef0_hbm, ef1_hbm, ef2_hbm, ef3_hbm, col3_hbm, ew_hbm, deg_hbm,
                 f0_v, f1_v, f2_v, f3_v, ew_v, col_v, z_v, deg_sp):
  cid = lax.axis_index("c")
  sid = lax.axis_index("s")
  wid = _wid()
  base = pl.multiple_of(wid * EPW, EPW)

  def zb(i, _):
    z_v[pl.ds(i * 16, 16)] = jnp.zeros((16,), jnp.float32)
    return 0
  lax.fori_loop(0, RPS // 16, zb, 0)
  pltpu.sync_copy(z_v, deg_sp.at[pl.ds(sid * RPS, RPS)])

  pltpu.sync_copy(col3_hbm.at[wid], col_v)
  pltpu.sync_copy(ef0_hbm.at[pl.ds(base, EPW)], f0_v)
  pltpu.sync_copy(ef1_hbm.at[pl.ds(base, EPW)], f1_v)
  pltpu.sync_copy(ef2_hbm.at[pl.ds(base, EPW)], f2_v)
  pltpu.sync_copy(ef3_hbm.at[pl.ds(base, EPW)], f3_v)

  def ewb(t, _):
    e0 = t * 16
    s = (f0_v[pl.ds(e0, 16)] + f1_v[pl.ds(e0, 16)]
         + f2_v[pl.ds(e0, 16)] + f3_v[pl.ds(e0, 16)])
    ew_v[pl.ds(e0, 16)] = s * 0.25
    return 0
  lax.fori_loop(0, EPW // 16, ewb, 0)
  pltpu.sync_copy(ew_v, ew_hbm.at[pl.ds(base, EPW)])

  plsc.subcore_barrier()

  def db(j, _):
    pltpu.sync_copy(ew_v.at[pl.ds(pl.multiple_of(j * CHUNK, CHUNK), CHUNK)],
                    deg_sp.at[col_v.at[j]], add=True)
    return 0
  lax.fori_loop(0, NCHUNK, db, 0)

  plsc.subcore_barrier()
  pltpu.sync_copy(deg_sp.at[pl.ds(sid * RPS, RPS)],
                  deg_hbm.at[cid, pl.ds(sid * RPS, RPS)])


@functools.cache
def _ew_deg():
  return pl.kernel(
    _ew_deg_body,
    out_type=(jax.ShapeDtypeStruct((E,), jnp.float32),
              jax.ShapeDtypeStruct((NC, NPAD), jnp.float32)),
    mesh=_mesh(),
    scratch_types=[
        pltpu.VMEM((EPW,), jnp.float32),
        pltpu.VMEM((EPW,), jnp.float32),
        pltpu.VMEM((EPW,), jnp.float32),
        pltpu.VMEM((EPW,), jnp.float32),
        pltpu.VMEM((EPW,), jnp.float32),
        pltpu.VMEM((NCHUNK, CHUNK), jnp.int32),
        pltpu.VMEM((RPS,), jnp.float32),
        pltpu.VMEM_SHARED((NPAD,), jnp.float32),
    ],
    compiler_params=pltpu.CompilerParams(needs_layout_passes=False),
  )


def _agg_body(g_hbm, row_hbm, col_hbm, ew_hbm, acc_hbm, *refs):
  row_s = refs[0:4]
  col_s = refs[4:8]
  ew_s = refs[8:12]
  rows = refs[12:16]
  acc_sp = refs[16]
  isem = refs[17:21]
  gsem = refs[21:25]
  ssem = refs[25:29]
  cid = lax.axis_index("c")
  sid = lax.axis_index("s")
  wid = _wid()
  base = pl.multiple_of(wid * EPW, EPW)

  def zb(i, _):
    r = i // 8
    j = i % 8
    rows[0][r, pl.ds(j * 16, 16)] = jnp.zeros((16,), jnp.float32)
    return 0
  lax.fori_loop(0, CHUNK * 8, zb, 0)
  for k in range(RPS // CHUNK):
    pltpu.sync_copy(rows[0], acc_sp.at[pl.ds(sid * RPS + k * CHUNK, CHUNK)])
  plsc.subcore_barrier()

  def scale(t):
    @plsc.parallel_loop(0, CHUNK, unroll=8)
    def _(e):
      bc = plsc.load_gather(ew_s[t], [jnp.full((16,), e, jnp.int32)])
      for jj in range(D // 16):
        rows[t][e, pl.ds(jj * 16, 16)] = rows[t][e, pl.ds(jj * 16, 16)] * bc

  def idx_issue(j, t):
    off = pl.multiple_of(base + j * CHUNK, CHUNK)
    pltpu.async_copy(row_hbm.at[pl.ds(off, CHUNK)], row_s[t], isem[t])
    pltpu.async_copy(col_hbm.at[pl.ds(off, CHUNK)], col_s[t], isem[t])
    pltpu.async_copy(ew_hbm.at[pl.ds(off, CHUNK)], ew_s[t], isem[t])

  def idx_wait(t):
    pltpu.make_async_copy(row_hbm.at[pl.ds(base, CHUNK)], row_s[t],
                          isem[t]).wait()
    pltpu.make_async_copy(col_hbm.at[pl.ds(base, CHUNK)], col_s[t],
                          isem[t]).wait()
    pltpu.make_async_copy(ew_hbm.at[pl.ds(base, CHUNK)], ew_s[t],
                          isem[t]).wait()

  def gissue(j, t):
    pltpu.async_copy(g_hbm.at[row_s[t]], rows[t], gsem[t])

  def gwait(t):
    pltpu.make_async_copy(g_hbm.at[row_s[t]], rows[t], gsem[t]).wait()

  def sissue(t):
    pltpu.async_copy(rows[t], acc_sp.at[col_s[t]], ssem[t], add=True)

  def swait(t):
    pltpu.make_async_copy(rows[t], acc_sp.at[col_s[t]], ssem[t]).wait()

  idx_issue(0, 0)
  idx_issue(1, 1)
  idx_wait(0)
  gissue(0, 0)

  idx_wait(1)
  gissue(1, 1)
  gwait(0)
  scale(0)
  sissue(0)
  idx_issue(2, 2)
  idx_wait(2)
  gissue(2, 2)
  gwait(1)
  scale(1)
  sissue(1)
  idx_issue(3, 3)

  def quad(i, _):
    for u in range(4):
      j = 2 + 4 * i + u
      t = (2 + u) % 4
      t1 = (3 + u) % 4
      t2 = u
      idx_wait(t1)
      gissue(j + 1, t1)
      gwait(t)
      scale(t)
      sissue(t)
      swait(t2)
      idx_issue(j + 2, t2)
    return 0
  lax.fori_loop(0, (NCHUNK - 5) // 4, quad, 0)

  idx_wait(3)
  gissue(123, 3)
  gwait(2)
  scale(2)
  sissue(2)
  swait(0)
  idx_issue(124, 0)

  idx_wait(0)
  gissue(124, 0)
  gwait(3)
  scale(3)
  sissue(3)
  swait(1)

  gwait(0)
  scale(0)
  sissue(0)
  swait(2)
  swait(3)
  swait(0)

  plsc.subcore_barrier()
  for k in range(RPS // CHUNK):
    pltpu.sync_copy(acc_sp.at[pl.ds(sid * RPS + k * CHUNK, CHUNK)],
                    acc_hbm.at[cid, pl.ds(sid * RPS + k * CHUNK, CHUNK)])


@functools.cache
def _agg():
  return pl.kernel(
    _agg_body,
    out_type=jax.ShapeDtypeStruct((NC, NPAD, D), jnp.float32),
    mesh=_mesh(),
    scratch_types=(
        [pltpu.VMEM((CHUNK,), jnp.int32)] * 4
        + [pltpu.VMEM((CHUNK,), jnp.int32)] * 4
        + [pltpu.VMEM((CHUNK,), jnp.float32)] * 4
        + [pltpu.VMEM((CHUNK, D), jnp.float32)] * 4
        + [pltpu.VMEM_SHARED((NPAD, D), jnp.float32)]
        + [pltpu.SemaphoreType.DMA] * 12
    ),
    compiler_params=pltpu.CompilerParams(needs_layout_passes=False),
  )


BN = 1000


def _tc1_body(x_ref, w_ref, degp_ref, dis_ref, g_ref):
  d = degp_ref[0] + degp_ref[1] + 1.0
  dis = jnp.where(d > 0, lax.rsqrt(d), 0.0)
  dis_ref[...] = dis
  h = jnp.dot(x_ref[...], w_ref[...], preferred_element_type=jnp.float32)
  g_ref[...] = h * dis


def _tc1(x, w, degp):
  return pl.pallas_call(
      _tc1_body,
      grid=(N // BN,),
      in_specs=[
          pl.BlockSpec((BN, D), lambda i: (i, 0)),
          pl.BlockSpec((D, D), lambda i: (0, 0)),
          pl.BlockSpec((NC, BN, 1), lambda i: (0, i, 0)),
      ],
      out_specs=[
          pl.BlockSpec((BN, 1), lambda i: (i, 0)),
          pl.BlockSpec((BN, D), lambda i: (i, 0)),
      ],
      out_shape=[
          jax.ShapeDtypeStruct((N, 1), jnp.float32),
          jax.ShapeDtypeStruct((N, D), jnp.float32),
      ],
  )(x, w, degp)


def _tc_mid_body(accp_ref, g_ref, dis_ref, b_ref, w_ref, g2_ref):
  acc = accp_ref[0] + accp_ref[1]
  out = dis_ref[...] * (acc + g_ref[...]) + b_ref[...]
  h2 = jnp.maximum(out, 0.0)
  g2_ref[...] = jnp.dot(h2, w_ref[...],
                        preferred_element_type=jnp.float32) * dis_ref[...]


def _tc_mid(accp, g, dis, b, w):
  return pl.pallas_call(
      _tc_mid_body,
      grid=(N // BN,),
      in_specs=[
          pl.BlockSpec((NC, BN, D), lambda i: (0, i, 0)),
          pl.BlockSpec((BN, D), lambda i: (i, 0)),
          pl.BlockSpec((BN, 1), lambda i: (i, 0)),
          pl.BlockSpec((1, D), lambda i: (0, 0)),
          pl.BlockSpec((D, D), lambda i: (0, 0)),
      ],
      out_specs=pl.BlockSpec((BN, D), lambda i: (i, 0)),
      out_shape=jax.ShapeDtypeStruct((N, D), jnp.float32),
  )(accp, g, dis, b, w)


def _tc_fin_body(accp_ref, g_ref, dis_ref, b_ref, out_ref):
  acc = accp_ref[0] + accp_ref[1]
  out_ref[...] = dis_ref[...] * (acc + g_ref[...]) + b_ref[...]


def _tc_fin(accp, g, dis, b):
  return pl.pallas_call(
      _tc_fin_body,
      grid=(N // BN,),
      in_specs=[
          pl.BlockSpec((NC, BN, D), lambda i: (0, i, 0)),
          pl.BlockSpec((BN, D), lambda i: (i, 0)),
          pl.BlockSpec((BN, 1), lambda i: (i, 0)),
          pl.BlockSpec((1, D), lambda i: (0, 0)),
      ],
      out_specs=pl.BlockSpec((BN, D), lambda i: (i, 0)),
      out_shape=jax.ShapeDtypeStruct((N, D), jnp.float32),
  )(accp, g, dis, b)


def kernel(x, edge_index, edge_feature, W1, b1, W2, b2):
  row = edge_index[0]
  col = edge_index[1]
  col3 = col.reshape(NW, NCHUNK, CHUNK)
  eft = edge_feature.T

  ew, deg = _ew_deg()(eft[0], eft[1], eft[2], eft[3], col3)
  degp = deg[:, :N].reshape(NC, N, 1)
  dis, g1 = _tc1(x, W1, degp)

  acc1 = _agg()(g1, row, col, ew)
  g2 = _tc_mid(acc1, g1, dis, b1.reshape(1, D), W2)

  acc2 = _agg()(g2, row, col, ew)
  out = _tc_fin(acc2, g2, dis, b2.reshape(1, D))
  return out


# speedup vs baseline: 1.1742x; 1.0048x over previous
import functools

import jax
import jax.numpy as jnp
from jax import lax
from jax.experimental import pallas as pl
from jax.experimental.pallas import tpu as pltpu
from jax.experimental.pallas import tpu_sc as plsc

N = 10000
E = 320000
D = 128

NC = 2
NS = 16
NW = NC * NS
EPW = E // NW
CHUNK = 80
NCHUNK = EPW // CHUNK
NPAD = 10240
RPS = NPAD // NS

@functools.cache
def _mesh():
  return plsc.VectorSubcoreMesh(
      core_axis_name="c", subcore_axis_name="s", num_cores=NC, num_subcores=NS)


def _wid():
  return lax.axis_index("c") * NS + lax.axis_index("s")


def _ew_deg_body(ef0_hbm, ef1_hbm, ef2_hbm, ef3_hbm, col3_hbm, ew_hbm, deg_hbm,
                 f0_v, f1_v, f2_v, f3_v, ew_v, col_v, z_v, deg_sp):
  cid = lax.axis_index("c")
  sid = lax.axis_index("s")
  wid = _wid()
  base = pl.multiple_of(wid * EPW, EPW)

  def zb(i, _):
    z_v[pl.ds(i * 16, 16)] = jnp.zeros((16,), jnp.float32)
    return 0
  lax.fori_loop(0, RPS // 16, zb, 0)
  pltpu.sync_copy(z_v, deg_sp.at[pl.ds(sid * RPS, RPS)])

  pltpu.sync_copy(col3_hbm.at[wid], col_v)
  pltpu.sync_copy(ef0_hbm.at[pl.ds(base, EPW)], f0_v)
  pltpu.sync_copy(ef1_hbm.at[pl.ds(base, EPW)], f1_v)
  pltpu.sync_copy(ef2_hbm.at[pl.ds(base, EPW)], f2_v)
  pltpu.sync_copy(ef3_hbm.at[pl.ds(base, EPW)], f3_v)

  def ewb(t, _):
    e0 = t * 16
    s = (f0_v[pl.ds(e0, 16)] + f1_v[pl.ds(e0, 16)]
         + f2_v[pl.ds(e0, 16)] + f3_v[pl.ds(e0, 16)])
    ew_v[pl.ds(e0, 16)] = s * 0.25
    return 0
  lax.fori_loop(0, EPW // 16, ewb, 0)
  pltpu.sync_copy(ew_v, ew_hbm.at[pl.ds(base, EPW)])

  plsc.subcore_barrier()

  def db(j, _):
    pltpu.sync_copy(ew_v.at[pl.ds(pl.multiple_of(j * CHUNK, CHUNK), CHUNK)],
                    deg_sp.at[col_v.at[j]], add=True)
    return 0
  lax.fori_loop(0, NCHUNK, db, 0)

  plsc.subcore_barrier()
  pltpu.sync_copy(deg_sp.at[pl.ds(sid * RPS, RPS)],
                  deg_hbm.at[cid, pl.ds(sid * RPS, RPS)])


@functools.cache
def _ew_deg():
  return pl.kernel(
    _ew_deg_body,
    out_type=(jax.ShapeDtypeStruct((E,), jnp.float32),
              jax.ShapeDtypeStruct((NC, NPAD), jnp.float32)),
    mesh=_mesh(),
    scratch_types=[
        pltpu.VMEM((EPW,), jnp.float32),
        pltpu.VMEM((EPW,), jnp.float32),
        pltpu.VMEM((EPW,), jnp.float32),
        pltpu.VMEM((EPW,), jnp.float32),
        pltpu.VMEM((EPW,), jnp.float32),
        pltpu.VMEM((NCHUNK, CHUNK), jnp.int32),
        pltpu.VMEM((RPS,), jnp.float32),
        pltpu.VMEM_SHARED((NPAD,), jnp.float32),
    ],
    compiler_params=pltpu.CompilerParams(needs_layout_passes=False),
  )


def _agg_body(g_hbm, row_hbm, col_hbm, ew_hbm, acc_hbm, *refs):
  row_s = refs[0:4]
  col_s = refs[4:8]
  ew_s = refs[8:12]
  rows = refs[12:16]
  acc_sp = refs[16]
  isem = refs[17:21]
  gsem = refs[21:25]
  ssem = refs[25:29]
  cid = lax.axis_index("c")
  sid = lax.axis_index("s")
  wid = _wid()
  base = pl.multiple_of(wid * EPW, EPW)

  def zb(i, _):
    r = i // 8
    j = i % 8
    rows[0][r, pl.ds(j * 16, 16)] = jnp.zeros((16,), jnp.float32)
    return 0
  lax.fori_loop(0, CHUNK * 8, zb, 0)
  for k in range(RPS // CHUNK):
    pltpu.sync_copy(rows[0], acc_sp.at[pl.ds(sid * RPS + k * CHUNK, CHUNK)])
  plsc.subcore_barrier()

  def scale(t):
    @plsc.parallel_loop(0, CHUNK, unroll=4)
    def _(e):
      bc = plsc.load_gather(ew_s[t], [jnp.full((16,), e, jnp.int32)])
      for jj in range(D // 16):
        rows[t][e, pl.ds(jj * 16, 16)] = rows[t][e, pl.ds(jj * 16, 16)] * bc

  def idx_issue(j, t):
    off = pl.multiple_of(base + j * CHUNK, CHUNK)
    pltpu.async_copy(row_hbm.at[pl.ds(off, CHUNK)], row_s[t], isem[t])
    pltpu.async_copy(col_hbm.at[pl.ds(off, CHUNK)], col_s[t], isem[t])
    pltpu.async_copy(ew_hbm.at[pl.ds(off, CHUNK)], ew_s[t], isem[t])

  def idx_wait(t):
    pltpu.make_async_copy(row_hbm.at[pl.ds(base, CHUNK)], row_s[t],
                          isem[t]).wait()
    pltpu.make_async_copy(col_hbm.at[pl.ds(base, CHUNK)], col_s[t],
                          isem[t]).wait()
    pltpu.make_async_copy(ew_hbm.at[pl.ds(base, CHUNK)], ew_s[t],
                          isem[t]).wait()

  def gissue(j, t):
    pltpu.async_copy(g_hbm.at[row_s[t]], rows[t], gsem[t])

  def gwait(t):
    pltpu.make_async_copy(g_hbm.at[row_s[t]], rows[t], gsem[t]).wait()

  def sissue(t):
    pltpu.async_copy(rows[t], acc_sp.at[col_s[t]], ssem[t], add=True)

  def swait(t):
    pltpu.make_async_copy(rows[t], acc_sp.at[col_s[t]], ssem[t]).wait()

  idx_issue(0, 0)
  idx_issue(1, 1)
  idx_wait(0)
  gissue(0, 0)

  idx_wait(1)
  gissue(1, 1)
  gwait(0)
  scale(0)
  sissue(0)
  idx_issue(2, 2)
  idx_wait(2)
  gissue(2, 2)
  gwait(1)
  scale(1)
  sissue(1)
  idx_issue(3, 3)

  def quad(i, _):
    for u in range(4):
      j = 2 + 4 * i + u
      t = (2 + u) % 4
      t1 = (3 + u) % 4
      t2 = u
      idx_wait(t1)
      gissue(j + 1, t1)
      gwait(t)
      scale(t)
      sissue(t)
      swait(t2)
      idx_issue(j + 2, t2)
    return 0
  lax.fori_loop(0, (NCHUNK - 5) // 4, quad, 0)

  idx_wait(3)
  gissue(123, 3)
  gwait(2)
  scale(2)
  sissue(2)
  swait(0)
  idx_issue(124, 0)

  idx_wait(0)
  gissue(124, 0)
  gwait(3)
  scale(3)
  sissue(3)
  swait(1)

  gwait(0)
  scale(0)
  sissue(0)
  swait(2)
  swait(3)
  swait(0)

  plsc.subcore_barrier()
  for k in range(RPS // CHUNK):
    pltpu.sync_copy(acc_sp.at[pl.ds(sid * RPS + k * CHUNK, CHUNK)],
                    acc_hbm.at[cid, pl.ds(sid * RPS + k * CHUNK, CHUNK)])


@functools.cache
def _agg():
  return pl.kernel(
    _agg_body,
    out_type=jax.ShapeDtypeStruct((NC, NPAD, D), jnp.float32),
    mesh=_mesh(),
    scratch_types=(
        [pltpu.VMEM((CHUNK,), jnp.int32)] * 4
        + [pltpu.VMEM((CHUNK,), jnp.int32)] * 4
        + [pltpu.VMEM((CHUNK,), jnp.float32)] * 4
        + [pltpu.VMEM((CHUNK, D), jnp.float32)] * 4
        + [pltpu.VMEM_SHARED((NPAD, D), jnp.float32)]
        + [pltpu.SemaphoreType.DMA] * 12
    ),
    compiler_params=pltpu.CompilerParams(needs_layout_passes=False),
  )


BN = 1000


def _tc1_body(x_ref, w_ref, degp_ref, dis_ref, g_ref):
  d = degp_ref[0] + degp_ref[1] + 1.0
  dis = jnp.where(d > 0, lax.rsqrt(d), 0.0)
  dis_ref[...] = dis
  h = jnp.dot(x_ref[...], w_ref[...], preferred_element_type=jnp.float32)
  g_ref[...] = h * dis


def _tc1(x, w, degp):
  return pl.pallas_call(
      _tc1_body,
      grid=(N // BN,),
      in_specs=[
          pl.BlockSpec((BN, D), lambda i: (i, 0)),
          pl.BlockSpec((D, D), lambda i: (0, 0)),
          pl.BlockSpec((NC, BN, 1), lambda i: (0, i, 0)),
      ],
      out_specs=[
          pl.BlockSpec((BN, 1), lambda i: (i, 0)),
          pl.BlockSpec((BN, D), lambda i: (i, 0)),
      ],
      out_shape=[
          jax.ShapeDtypeStruct((N, 1), jnp.float32),
          jax.ShapeDtypeStruct((N, D), jnp.float32),
      ],
  )(x, w, degp)


def _tc_mid_body(accp_ref, g_ref, dis_ref, b_ref, w_ref, g2_ref):
  acc = accp_ref[0] + accp_ref[1]
  out = dis_ref[...] * (acc + g_ref[...]) + b_ref[...]
  h2 = jnp.maximum(out, 0.0)
  g2_ref[...] = jnp.dot(h2, w_ref[...],
                        preferred_element_type=jnp.float32) * dis_ref[...]


def _tc_mid(accp, g, dis, b, w):
  return pl.pallas_call(
      _tc_mid_body,
      grid=(N // BN,),
      in_specs=[
          pl.BlockSpec((NC, BN, D), lambda i: (0, i, 0)),
          pl.BlockSpec((BN, D), lambda i: (i, 0)),
          pl.BlockSpec((BN, 1), lambda i: (i, 0)),
          pl.BlockSpec((1, D), lambda i: (0, 0)),
          pl.BlockSpec((D, D), lambda i: (0, 0)),
      ],
      out_specs=pl.BlockSpec((BN, D), lambda i: (i, 0)),
      out_shape=jax.ShapeDtypeStruct((N, D), jnp.float32),
  )(accp, g, dis, b, w)


def _tc_fin_body(accp_ref, g_ref, dis_ref, b_ref, out_ref):
  acc = accp_ref[0] + accp_ref[1]
  out_ref[...] = dis_ref[...] * (acc + g_ref[...]) + b_ref[...]


def _tc_fin(accp, g, dis, b):
  return pl.pallas_call(
      _tc_fin_body,
      grid=(N // BN,),
      in_specs=[
          pl.BlockSpec((NC, BN, D), lambda i: (0, i, 0)),
          pl.BlockSpec((BN, D), lambda i: (i, 0)),
          pl.BlockSpec((BN, 1), lambda i: (i, 0)),
          pl.BlockSpec((1, D), lambda i: (0, 0)),
      ],
      out_specs=pl.BlockSpec((BN, D), lambda i: (i, 0)),
      out_shape=jax.ShapeDtypeStruct((N, D), jnp.float32),
  )(accp, g, dis, b)


def kernel(x, edge_index, edge_feature, W1, b1, W2, b2):
  row = edge_index[0]
  col = edge_index[1]
  col3 = col.reshape(NW, NCHUNK, CHUNK)
  eft = edge_feature.T

  ew, deg = _ew_deg()(eft[0], eft[1], eft[2], eft[3], col3)
  degp = deg[:, :N].reshape(NC, N, 1)
  dis, g1 = _tc1(x, W1, degp)

  acc1 = _agg()(g1, row, col, ew)
  g2 = _tc_mid(acc1, g1, dis, b1.reshape(1, D), W2)

  acc2 = _agg()(g2, row, col, ew)
  out = _tc_fin(acc2, g2, dis, b2.reshape(1, D))
  return out


# speedup vs baseline: 1.2437x; 1.0591x over previous
import functools

import jax
import jax.numpy as jnp
from jax import lax
from jax.experimental import pallas as pl
from jax.experimental.pallas import tpu as pltpu
from jax.experimental.pallas import tpu_sc as plsc

N = 10000
E = 320000
D = 128

NC = 2
NS = 16
NW = NC * NS
EPW = E // NW
CHUNK = 80
NCHUNK = EPW // CHUNK
NPAD = 10240
RPS = NPAD // NS

@functools.cache
def _mesh():
  return plsc.VectorSubcoreMesh(
      core_axis_name="c", subcore_axis_name="s", num_cores=NC, num_subcores=NS)


def _wid():
  return lax.axis_index("c") * NS + lax.axis_index("s")


def _ew_deg_body(ew_hbm, col3_hbm, deg_hbm, ew_v, col_v, z_v, deg_sp):
  cid = lax.axis_index("c")
  sid = lax.axis_index("s")
  wid = _wid()
  base = pl.multiple_of(wid * EPW, EPW)

  def zb(i, _):
    z_v[pl.ds(i * 16, 16)] = jnp.zeros((16,), jnp.float32)
    return 0
  lax.fori_loop(0, RPS // 16, zb, 0)
  pltpu.sync_copy(z_v, deg_sp.at[pl.ds(sid * RPS, RPS)])

  pltpu.sync_copy(col3_hbm.at[wid], col_v)
  pltpu.sync_copy(ew_hbm.at[pl.ds(base, EPW)], ew_v)

  plsc.subcore_barrier()

  def db(j, _):
    pltpu.sync_copy(ew_v.at[pl.ds(pl.multiple_of(j * CHUNK, CHUNK), CHUNK)],
                    deg_sp.at[col_v.at[j]], add=True)
    return 0
  lax.fori_loop(0, NCHUNK, db, 0)

  plsc.subcore_barrier()
  pltpu.sync_copy(deg_sp.at[pl.ds(sid * RPS, RPS)],
                  deg_hbm.at[cid, pl.ds(sid * RPS, RPS)])


@functools.cache
def _ew_deg():
  return pl.kernel(
    _ew_deg_body,
    out_type=jax.ShapeDtypeStruct((NC, NPAD), jnp.float32),
    mesh=_mesh(),
    scratch_types=[
        pltpu.VMEM((EPW,), jnp.float32),
        pltpu.VMEM((NCHUNK, CHUNK), jnp.int32),
        pltpu.VMEM((RPS,), jnp.float32),
        pltpu.VMEM_SHARED((NPAD,), jnp.float32),
    ],
    compiler_params=pltpu.CompilerParams(needs_layout_passes=False),
  )


def _tc_ew_body(eft_ref, ew_ref):
  ew_ref[...] = (eft_ref[0] + eft_ref[1] + eft_ref[2] + eft_ref[3]) * 0.25


def _tc_ew(eft):
  return pl.pallas_call(
      _tc_ew_body,
      out_shape=jax.ShapeDtypeStruct((E,), jnp.float32),
  )(eft)


def _agg_body(g_hbm, row_hbm, col_hbm, ew_hbm, acc_hbm, *refs):
  row_s = refs[0:4]
  col_s = refs[4:8]
  ew_s = refs[8:12]
  rows = refs[12:16]
  acc_sp = refs[16]
  isem = refs[17:21]
  gsem = refs[21:25]
  ssem = refs[25:29]
  cid = lax.axis_index("c")
  sid = lax.axis_index("s")
  wid = _wid()
  base = pl.multiple_of(wid * EPW, EPW)

  def zb(i, _):
    r = i // 8
    j = i % 8
    rows[0][r, pl.ds(j * 16, 16)] = jnp.zeros((16,), jnp.float32)
    return 0
  lax.fori_loop(0, CHUNK * 8, zb, 0)
  for k in range(RPS // CHUNK):
    pltpu.sync_copy(rows[0], acc_sp.at[pl.ds(sid * RPS + k * CHUNK, CHUNK)])
  plsc.subcore_barrier()

  def scale(t):
    @plsc.parallel_loop(0, CHUNK, unroll=4)
    def _(e):
      bc = plsc.load_gather(ew_s[t], [jnp.full((16,), e, jnp.int32)])
      for jj in range(D // 16):
        rows[t][e, pl.ds(jj * 16, 16)] = rows[t][e, pl.ds(jj * 16, 16)] * bc

  def idx_issue(j, t):
    off = pl.multiple_of(base + j * CHUNK, CHUNK)
    pltpu.async_copy(row_hbm.at[pl.ds(off, CHUNK)], row_s[t], isem[t])
    pltpu.async_copy(col_hbm.at[pl.ds(off, CHUNK)], col_s[t], isem[t])
    pltpu.async_copy(ew_hbm.at[pl.ds(off, CHUNK)], ew_s[t], isem[t])

  def idx_wait(t):
    pltpu.make_async_copy(row_hbm.at[pl.ds(base, CHUNK)], row_s[t],
                          isem[t]).wait()
    pltpu.make_async_copy(col_hbm.at[pl.ds(base, CHUNK)], col_s[t],
                          isem[t]).wait()
    pltpu.make_async_copy(ew_hbm.at[pl.ds(base, CHUNK)], ew_s[t],
                          isem[t]).wait()

  def gissue(j, t):
    pltpu.async_copy(g_hbm.at[row_s[t]], rows[t], gsem[t])

  def gwait(t):
    pltpu.make_async_copy(g_hbm.at[row_s[t]], rows[t], gsem[t]).wait()

  def sissue(t):
    pltpu.async_copy(rows[t], acc_sp.at[col_s[t]], ssem[t], add=True)

  def swait(t):
    pltpu.make_async_copy(rows[t], acc_sp.at[col_s[t]], ssem[t]).wait()

  idx_issue(0, 0)
  idx_issue(1, 1)
  idx_wait(0)
  gissue(0, 0)

  idx_wait(1)
  gissue(1, 1)
  gwait(0)
  scale(0)
  sissue(0)
  idx_issue(2, 2)
  idx_wait(2)
  gissue(2, 2)
  gwait(1)
  scale(1)
  sissue(1)
  idx_issue(3, 3)

  def quad(i, _):
    for u in range(4):
      j = 2 + 4 * i + u
      t = (2 + u) % 4
      t1 = (3 + u) % 4
      t2 = u
      idx_wait(t1)
      gissue(j + 1, t1)
      gwait(t)
      scale(t)
      sissue(t)
      swait(t2)
      idx_issue(j + 2, t2)
    return 0
  lax.fori_loop(0, (NCHUNK - 5) // 4, quad, 0)

  idx_wait(3)
  gissue(123, 3)
  gwait(2)
  scale(2)
  sissue(2)
  swait(0)
  idx_issue(124, 0)

  idx_wait(0)
  gissue(124, 0)
  gwait(3)
  scale(3)
  sissue(3)
  swait(1)

  gwait(0)
  scale(0)
  sissue(0)
  swait(2)
  swait(3)
  swait(0)

  plsc.subcore_barrier()
  for k in range(RPS // CHUNK):
    pltpu.sync_copy(acc_sp.at[pl.ds(sid * RPS + k * CHUNK, CHUNK)],
                    acc_hbm.at[cid, pl.ds(sid * RPS + k * CHUNK, CHUNK)])


@functools.cache
def _agg():
  return pl.kernel(
    _agg_body,
    out_type=jax.ShapeDtypeStruct((NC, NPAD, D), jnp.float32),
    mesh=_mesh(),
    scratch_types=(
        [pltpu.VMEM((CHUNK,), jnp.int32)] * 4
        + [pltpu.VMEM((CHUNK,), jnp.int32)] * 4
        + [pltpu.VMEM((CHUNK,), jnp.float32)] * 4
        + [pltpu.VMEM((CHUNK, D), jnp.float32)] * 4
        + [pltpu.VMEM_SHARED((NPAD, D), jnp.float32)]
        + [pltpu.SemaphoreType.DMA] * 12
    ),
    compiler_params=pltpu.CompilerParams(needs_layout_passes=False),
  )


BN = 1000


def _tc1_body(x_ref, w_ref, degp_ref, dis_ref, g_ref):
  d = degp_ref[0] + degp_ref[1] + 1.0
  dis = jnp.where(d > 0, lax.rsqrt(d), 0.0)
  dis_ref[...] = dis
  h = jnp.dot(x_ref[...], w_ref[...], preferred_element_type=jnp.float32)
  g_ref[...] = h * dis


def _tc1(x, w, degp):
  return pl.pallas_call(
      _tc1_body,
      grid=(N // BN,),
      in_specs=[
          pl.BlockSpec((BN, D), lambda i: (i, 0)),
          pl.BlockSpec((D, D), lambda i: (0, 0)),
          pl.BlockSpec((NC, BN, 1), lambda i: (0, i, 0)),
      ],
      out_specs=[
          pl.BlockSpec((BN, 1), lambda i: (i, 0)),
          pl.BlockSpec((BN, D), lambda i: (i, 0)),
      ],
      out_shape=[
          jax.ShapeDtypeStruct((N, 1), jnp.float32),
          jax.ShapeDtypeStruct((N, D), jnp.float32),
      ],
  )(x, w, degp)


def _tc_mid_body(accp_ref, g_ref, dis_ref, b_ref, w_ref, g2_ref):
  acc = accp_ref[0] + accp_ref[1]
  out = dis_ref[...] * (acc + g_ref[...]) + b_ref[...]
  h2 = jnp.maximum(out, 0.0)
  g2_ref[...] = jnp.dot(h2, w_ref[...],
                        preferred_element_type=jnp.float32) * dis_ref[...]


def _tc_mid(accp, g, dis, b, w):
  return pl.pallas_call(
      _tc_mid_body,
      grid=(N // BN,),
      in_specs=[
          pl.BlockSpec((NC, BN, D), lambda i: (0, i, 0)),
          pl.BlockSpec((BN, D), lambda i: (i, 0)),
          pl.BlockSpec((BN, 1), lambda i: (i, 0)),
          pl.BlockSpec((1, D), lambda i: (0, 0)),
          pl.BlockSpec((D, D), lambda i: (0, 0)),
      ],
      out_specs=pl.BlockSpec((BN, D), lambda i: (i, 0)),
      out_shape=jax.ShapeDtypeStruct((N, D), jnp.float32),
  )(accp, g, dis, b, w)


def _tc_fin_body(accp_ref, g_ref, dis_ref, b_ref, out_ref):
  acc = accp_ref[0] + accp_ref[1]
  out_ref[...] = dis_ref[...] * (acc + g_ref[...]) + b_ref[...]


def _tc_fin(accp, g, dis, b):
  return pl.pallas_call(
      _tc_fin_body,
      grid=(N // BN,),
      in_specs=[
          pl.BlockSpec((NC, BN, D), lambda i: (0, i, 0)),
          pl.BlockSpec((BN, D), lambda i: (i, 0)),
          pl.BlockSpec((BN, 1), lambda i: (i, 0)),
          pl.BlockSpec((1, D), lambda i: (0, 0)),
      ],
      out_specs=pl.BlockSpec((BN, D), lambda i: (i, 0)),
      out_shape=jax.ShapeDtypeStruct((N, D), jnp.float32),
  )(accp, g, dis, b)


def kernel(x, edge_index, edge_feature, W1, b1, W2, b2):
  row = edge_index[0]
  col = edge_index[1]
  col3 = col.reshape(NW, NCHUNK, CHUNK)
  eft = edge_feature.T

  ew = _tc_ew(eft)
  deg = _ew_deg()(ew, col3)
  degp = deg[:, :N].reshape(NC, N, 1)
  dis, g1 = _tc1(x, W1, degp)

  acc1 = _agg()(g1, row, col, ew)
  g2 = _tc_mid(acc1, g1, dis, b1.reshape(1, D), W2)

  acc2 = _agg()(g2, row, col, ew)
  out = _tc_fin(acc2, g2, dis, b2.reshape(1, D))
  return out
